# baseline (speedup 1.0000x reference)
import functools

import jax
import jax.numpy as jnp
from jax import lax
from jax.experimental import pallas as pl
from jax.experimental.pallas import tpu as pltpu
from jax.experimental.pallas import tpu_sc as plsc

V = 100000
F = 26
D = 32
B = 4096
NUM = 13
EPS = 1e-5

NC = 2
NS = 16
NW = NC * NS
ROWS = F * D
RPT = ROWS // NW


def _sc_gather(xcatT, tt):
    mesh = plsc.VectorSubcoreMesh(core_axis_name="c", subcore_axis_name="s")

    HA = 51200
    HB = V - HA

    @functools.partial(
        pl.kernel,
        mesh=mesh,
        out_type=jax.ShapeDtypeStruct((ROWS, B), jnp.float32),
        scratch_types=[
            pltpu.VMEM((B,), jnp.int32),
            pltpu.VMEM((HA,), jnp.float32),
            pltpu.VMEM((HB,), jnp.float32),
            pltpu.VMEM((2 * B,), jnp.float32),
            pltpu.SemaphoreType.DMA,
            pltpu.SemaphoreType.DMA,
            pltpu.SemaphoreType.DMA,
        ],
        compiler_params=pltpu.CompilerParams(needs_layout_passes=False),
    )
    def gather_k(xcatT_hbm, tt_hbm, g_hbm, idx_b, buf_a, buf_b, stage2,
                 sem_a, sem_b, sem_o):
        wid = lax.axis_index("s") * NC + lax.axis_index("c")
        base = wid * RPT

        def fd(j):
            r = base + j
            return r, r // D, lax.rem(r, D)

        def start_a(j):
            _, f, d = fd(j)
            pltpu.async_copy(tt_hbm.at[f, d, pl.ds(0, HA)], buf_a, sem_a)

        def start_b(j):
            _, f, d = fd(j)
            pltpu.async_copy(tt_hbm.at[f, d, pl.ds(HA, HB)], buf_b, sem_b)

        start_a(0)
        start_b(0)

        def row_body(j, f_prev):
            r, f, d = fd(j)
            par = lax.rem(j, 2) * B

            @pl.when(j >= 2)
            def _():
                r2 = r - 2
                pltpu.make_async_copy(
                    stage2.at[pl.ds(par, B)], g_hbm.at[r2], sem_o).wait()

            @pl.when(f != f_prev)
            def _():
                pltpu.sync_copy(xcatT_hbm.at[f], idx_b)

            pltpu.make_async_copy(
                tt_hbm.at[f, d, pl.ds(0, HA)], buf_a, sem_a).wait()

            def inner_a(k, c):
                vv = idx_b[pl.ds(k * 16, 16)]
                vv = jnp.minimum(jnp.maximum(vv, 0), HA - 1)
                stage2[pl.ds(par + k * 16, 16)] = plsc.load_gather(buf_a, [vv])
                return c

            lax.fori_loop(0, B // 16, inner_a, 0)

            @pl.when(j < RPT - 1)
            def _():
                start_a(j + 1)

            pltpu.make_async_copy(
                tt_hbm.at[f, d, pl.ds(HA, HB)], buf_b, sem_b).wait()

            def inner_b(k, c):
                vv = idx_b[pl.ds(k * 16, 16)]
                hi = vv >= HA
                loc = jnp.minimum(jnp.maximum(vv - HA, 0), HB - 1)
                g = plsc.load_gather(buf_b, [loc])
                prev = stage2[pl.ds(par + k * 16, 16)]
                stage2[pl.ds(par + k * 16, 16)] = jnp.where(hi, g, prev)
                return c

            lax.fori_loop(0, B // 16, inner_b, 0)

            @pl.when(j < RPT - 1)
            def _():
                start_b(j + 1)

            pltpu.async_copy(stage2.at[pl.ds(par, B)], g_hbm.at[r], sem_o)
            return f

        lax.fori_loop(0, RPT, row_body, -1)

        for jj in (RPT - 2, RPT - 1):
            pltpu.make_async_copy(
                stage2.at[pl.ds(lax.rem(jj, 2) * B, B)],
                g_hbm.at[base + jj], sem_o).wait()

    return gather_k(xcatT, tt)


def _bn_relu(y, g, bt):
    m = jnp.mean(y, axis=0, keepdims=True)
    c = y - m
    v = jnp.mean(c * c, axis=0, keepdims=True)
    return jnp.maximum(c * lax.rsqrt(v + EPS) * g + bt, 0.0)


def _mlp_body(g_ref, xnum_ref, w1a, w1b, b1, g1, bt1, w2, b2, g2, bt2,
              w3, b3, g3, bt3, w4, b4, out_ref):
    y = lax.dot_general(g_ref[...], w1a[...], (((0,), (0,)), ((), ())),
                        preferred_element_type=jnp.float32)
    y = y + jnp.dot(xnum_ref[...], w1b[...], preferred_element_type=jnp.float32)
    h = _bn_relu(y + b1[...], g1[...], bt1[...])
    y = jnp.dot(h, w2[...], preferred_element_type=jnp.float32) + b2[...]
    h = _bn_relu(y, g2[...], bt2[...])
    y = jnp.dot(h, w3[...], preferred_element_type=jnp.float32) + b3[...]
    h = _bn_relu(y, g3[...], bt3[...])
    out_ref[...] = jnp.dot(h, w4[...], preferred_element_type=jnp.float32) + b4[...]


def _mlp(G, x_num, W1a, W1b, b1, g1, bt1, W2, b2, g2, bt2,
         W3, b3, g3, bt3, W4, b4):
    return pl.pallas_call(
        _mlp_body,
        out_shape=jax.ShapeDtypeStruct((B, 1), jnp.float32),
    )(G, x_num, W1a, W1b, b1, g1, bt1, W2, b2, g2, bt2,
      W3, b3, g3, bt3, W4, b4)


def kernel(x_cat, x_num, tables, W1, b1, g1, bt1, W2, b2, g2, bt2,
           W3, b3, g3, bt3, W4, b4):
    xcatT = x_cat.astype(jnp.int32).T
    tt = tables.transpose(0, 2, 1)
    G = _sc_gather(xcatT, tt)
    W1a = W1[: F * D]
    W1b = W1[F * D:]
    r = lambda a: a.reshape(1, -1)
    out = _mlp(G, x_num, W1a, W1b, r(b1), r(g1), r(bt1),
               W2, r(b2), r(g2), r(bt2), W3, r(b3), r(g3), r(bt3),
               W4, r(b4))
    return out.reshape(B)


# speedup vs baseline: 1.2126x; 1.2126x over previous
import functools

import jax
import jax.numpy as jnp
from jax import lax
from jax.experimental import pallas as pl
from jax.experimental.pallas import tpu as pltpu
from jax.experimental.pallas import tpu_sc as plsc

V = 100000
F = 26
D = 32
B = 4096
NUM = 13
EPS = 1e-5

NC = 2
NS = 16
NW = NC * NS
ROWS = F * D
RPT = ROWS // NW


def _sc_gather(xcatT, tt):
    mesh = plsc.VectorSubcoreMesh(core_axis_name="c", subcore_axis_name="s")

    HA = 51200
    HB = V - HA

    @functools.partial(
        pl.kernel,
        mesh=mesh,
        out_type=jax.ShapeDtypeStruct((ROWS, B), jnp.float32),
        scratch_types=[
            pltpu.VMEM((B,), jnp.int32),
            pltpu.VMEM((HA,), jnp.float32),
            pltpu.VMEM((HB,), jnp.float32),
            pltpu.VMEM((B,), jnp.float32),
            pltpu.SemaphoreType.DMA,
            pltpu.SemaphoreType.DMA,
        ],
        compiler_params=pltpu.CompilerParams(needs_layout_passes=False),
    )
    def gather_k(xcatT_hbm, tt_hbm, g_hbm, idx_b, buf_a, buf_b, stage_b,
                 sem_a, sem_b):
        wid = lax.axis_index("s") * NC + lax.axis_index("c")
        base = wid * RPT

        def fd(j):
            r = base + j
            return r, r // D, lax.rem(r, D)

        def start_a(j):
            _, f, d = fd(j)
            pltpu.async_copy(tt_hbm.at[f, d, pl.ds(0, HA)], buf_a, sem_a)

        def start_b(j):
            _, f, d = fd(j)
            pltpu.async_copy(tt_hbm.at[f, d, pl.ds(HA, HB)], buf_b, sem_b)

        start_a(0)
        start_b(0)

        def row_body(j, f_prev):
            r, f, d = fd(j)

            @pl.when(f != f_prev)
            def _():
                pltpu.sync_copy(xcatT_hbm.at[f], idx_b)

            pltpu.make_async_copy(
                tt_hbm.at[f, d, pl.ds(0, HA)], buf_a, sem_a).wait()

            def inner_a(k, c):
                vv = idx_b[pl.ds(k * 16, 16)]
                vv = jnp.minimum(jnp.maximum(vv, 0), HA - 1)
                stage_b[pl.ds(k * 16, 16)] = plsc.load_gather(buf_a, [vv])
                return c

            lax.fori_loop(0, B // 16, inner_a, 0)

            @pl.when(j < RPT - 1)
            def _():
                start_a(j + 1)

            pltpu.make_async_copy(
                tt_hbm.at[f, d, pl.ds(HA, HB)], buf_b, sem_b).wait()

            def inner_b(k, c):
                vv = idx_b[pl.ds(k * 16, 16)]
                hi = vv >= HA
                loc = jnp.minimum(jnp.maximum(vv - HA, 0), HB - 1)
                g = plsc.load_gather(buf_b, [loc])
                prev = stage_b[pl.ds(k * 16, 16)]
                stage_b[pl.ds(k * 16, 16)] = jnp.where(hi, g, prev)
                return c

            lax.fori_loop(0, B // 16, inner_b, 0)

            @pl.when(j < RPT - 1)
            def _():
                start_b(j + 1)

            pltpu.sync_copy(stage_b, g_hbm.at[r])
            return f

        lax.fori_loop(0, RPT, row_body, -1)

    return gather_k(xcatT, tt)


def _bn_relu(y, g, bt):
    m = jnp.mean(y, axis=0, keepdims=True)
    c = y - m
    v = jnp.mean(c * c, axis=0, keepdims=True)
    return jnp.maximum(c * lax.rsqrt(v + EPS) * g + bt, 0.0)


def _mlp_body(g_ref, xnum_ref, w1a, w1b, b1, g1, bt1, w2, b2, g2, bt2,
              w3, b3, g3, bt3, w4, b4, out_ref):
    y = lax.dot_general(g_ref[...], w1a[...], (((0,), (0,)), ((), ())),
                        preferred_element_type=jnp.float32)
    y = y + jnp.dot(xnum_ref[...], w1b[...], preferred_element_type=jnp.float32)
    h = _bn_relu(y + b1[...], g1[...], bt1[...])
    y = jnp.dot(h, w2[...], preferred_element_type=jnp.float32) + b2[...]
    h = _bn_relu(y, g2[...], bt2[...])
    y = jnp.dot(h, w3[...], preferred_element_type=jnp.float32) + b3[...]
    h = _bn_relu(y, g3[...], bt3[...])
    out_ref[...] = jnp.dot(h, w4[...], preferred_element_type=jnp.float32) + b4[...]


def _mlp(G, x_num, W1a, W1b, b1, g1, bt1, W2, b2, g2, bt2,
         W3, b3, g3, bt3, W4, b4):
    return pl.pallas_call(
        _mlp_body,
        out_shape=jax.ShapeDtypeStruct((B, 1), jnp.float32),
    )(G, x_num, W1a, W1b, b1, g1, bt1, W2, b2, g2, bt2,
      W3, b3, g3, bt3, W4, b4)


def kernel(x_cat, x_num, tables, W1, b1, g1, bt1, W2, b2, g2, bt2,
           W3, b3, g3, bt3, W4, b4):
    xcatT = x_cat.astype(jnp.int32).T
    tt = tables.transpose(0, 2, 1)
    G = _sc_gather(xcatT, tt)
    W1a = W1[: F * D]
    W1b = W1[F * D:]
    r = lambda a: a.reshape(1, -1)
    out = _mlp(G, x_num, W1a, W1b, r(b1), r(g1), r(bt1),
               W2, r(b2), r(g2), r(bt2), W3, r(b3), r(g3), r(bt3),
               W4, r(b4))
    return out.reshape(B)
